# Initial kernel scaffold; baseline (speedup 1.0000x reference)
import jax
import jax.numpy as jnp
from jax import lax
from jax.experimental import pallas as pl
from jax.experimental.pallas import tpu as pltpu
from jax.experimental.pallas import tpu_sc as plsc

N_USER_C = 25000
N_NODES = 50000
EMB_C = 64
HALF = 32
NNZ_C = 800000
B_C = 4096

NTILES = 16
NCORES = 2
CHUNK = 128
CPT = 400
EBLK = 16
NBLK = CPT // EBLK
E_PAD = NTILES * CPT * CHUNK
ROWS_PER_TILE = N_NODES // NTILES
ZROWS = 125


def _spmm_body(egoH, cols2d, rows2d, vals2d, out_hbm,
               cols_blk, rows_blk, vals_blk, cidx, gbuf, accum, zrow,
               gsem):
    c = lax.axis_index("c")
    s = lax.axis_index("s")

    zv = jnp.zeros((16,), jnp.float32)

    def zfill(r, _):
        zrow[r, pl.ds(0, 16)] = zv
        zrow[r, pl.ds(16, 16)] = zv
        return 0

    lax.fori_loop(0, ZROWS, zfill, 0)

    def zloop(k, _):
        pltpu.sync_copy(zrow,
                        accum.at[pl.ds(s * ROWS_PER_TILE + k * ZROWS, ZROWS)])
        return 0

    lax.fori_loop(0, ROWS_PER_TILE // ZROWS, zloop, 0)
    plsc.subcore_barrier()

    coff = c * N_NODES

    def block(b, _):
        r0 = s * CPT + b * EBLK
        pltpu.sync_copy(cols2d.at[pl.ds(r0, EBLK)], cols_blk)
        pltpu.sync_copy(rows2d.at[pl.ds(r0, EBLK)], rows_blk)
        pltpu.sync_copy(vals2d.at[pl.ds(r0, EBLK)], vals_blk)

        def chunk(j, _):
            def adj(q, _):
                cidx[0, pl.ds(q * 16, 16)] = (
                    cols_blk[j, pl.ds(q * 16, 16)] + coff)
                return 0

            lax.fori_loop(0, CHUNK // 16, adj, 0)
            pltpu.async_copy(egoH.at[cidx.at[0]], gbuf, gsem).wait()

            def edge(e, _):
                bv = jnp.broadcast_to(vals_blk[j, e], (16,))
                gbuf[e, pl.ds(0, 16)] = gbuf[e, pl.ds(0, 16)] * bv
                gbuf[e, pl.ds(16, 16)] = gbuf[e, pl.ds(16, 16)] * bv
                return 0

            lax.fori_loop(0, CHUNK, edge, 0)
            pltpu.sync_copy(gbuf, accum.at[rows_blk.at[j]], add=True)
            return 0

        lax.fori_loop(0, EBLK, chunk, 0)
        return 0

    lax.fori_loop(0, NBLK, block, 0)

    plsc.subcore_barrier()
    pltpu.sync_copy(
        accum.at[pl.ds(s * ROWS_PER_TILE, ROWS_PER_TILE)],
        out_hbm.at[pl.ds(c * N_NODES + s * ROWS_PER_TILE, ROWS_PER_TILE)])


_spmm = pl.kernel(
    _spmm_body,
    out_type=jax.ShapeDtypeStruct((2 * N_NODES, HALF), jnp.float32),
    mesh=plsc.VectorSubcoreMesh(core_axis_name="c", subcore_axis_name="s"),
    scratch_types=[
        pltpu.VMEM((EBLK, CHUNK), jnp.int32),
        pltpu.VMEM((EBLK, CHUNK), jnp.int32),
        pltpu.VMEM((EBLK, CHUNK), jnp.float32),
        pltpu.VMEM((1, CHUNK), jnp.int32),
        pltpu.VMEM((CHUNK, HALF), jnp.float32),
        pltpu.VMEM_SHARED((N_NODES, HALF), jnp.float32),
        pltpu.VMEM((ZROWS, HALF), jnp.float32),
        pltpu.SemaphoreType.DMA,
    ],
)


def _dense_body(side_ref, ego_ref, wg_ref, bg_ref, wb_ref, bb_ref,
                ego_out, norm_out):
    side = side_ref[...]
    ego = ego_ref[...]
    se = jnp.dot(side, wg_ref[...],
                 preferred_element_type=jnp.float32) + bg_ref[0:1, :]
    bi = jnp.dot(ego * side, wb_ref[...],
                 preferred_element_type=jnp.float32) + bb_ref[0:1, :]
    x = se + bi
    x = jnp.where(x >= 0, x, 0.2 * x)
    nrm = jnp.sqrt(jnp.sum(x * x, axis=1, keepdims=True))
    norm_out[...] = x / jnp.maximum(nrm, 1e-12)
    ego_out[...] = x


def _dense(side, ego, wg, bg8, wb, bb8):
    R = 2000
    nb = N_NODES // R
    return pl.pallas_call(
        _dense_body,
        grid=(nb,),
        in_specs=[
            pl.BlockSpec((R, EMB_C), lambda i: (i, 0)),
            pl.BlockSpec((R, EMB_C), lambda i: (i, 0)),
            pl.BlockSpec((EMB_C, EMB_C), lambda i: (0, 0)),
            pl.BlockSpec((8, EMB_C), lambda i: (0, 0)),
            pl.BlockSpec((EMB_C, EMB_C), lambda i: (0, 0)),
            pl.BlockSpec((8, EMB_C), lambda i: (0, 0)),
        ],
        out_specs=[
            pl.BlockSpec((R, EMB_C), lambda i: (i, 0)),
            pl.BlockSpec((R, EMB_C), lambda i: (i, 0)),
        ],
        out_shape=[jax.ShapeDtypeStruct((N_NODES, EMB_C), jnp.float32)] * 2,
    )(side, ego, wg, bg8, wb, bb8)


PPT = B_C // (NTILES * NCORES)


def _score_body(t0, t1, t2, t3, uidx, iidx, wvec, out_hbm,
                uix, iix, ub0, ib0, ub1, ib1, ub2, ib2, ub3, ib3, wv, ob,
                sem):
    c = lax.axis_index("c")
    s = lax.axis_index("s")
    wid = s * NCORES + c
    base = wid * PPT
    pltpu.sync_copy(uidx.at[pl.ds(base, PPT)], uix)
    pltpu.sync_copy(iidx.at[pl.ds(base, PPT)], iix)
    pltpu.sync_copy(wvec, wv)
    bufs = [(ub0, ib0), (ub1, ib1), (ub2, ib2), (ub3, ib3)]
    descs = []
    for t, tab in enumerate([t0, t1, t2, t3]):
        descs.append(pltpu.async_copy(tab.at[uix], bufs[t][0], sem))
        descs.append(pltpu.async_copy(tab.at[iix], bufs[t][1], sem))
    for d in descs:
        d.wait()

    for g in range(PPT // 16):
        rowv = lax.iota(jnp.int32, 16) + g * 16
        acc = jnp.zeros((16,), jnp.float32)
        for t in range(4):
            ub, ib = bufs[t]

            def fbody(f, acc, ub=ub, ib=ib, t=t):
                colv = jnp.broadcast_to(f, (16,)).astype(jnp.int32)
                uvv = plsc.load_gather(ub, [rowv, colv])
                ivv = plsc.load_gather(ib, [rowv, colv])
                wf = jnp.broadcast_to(wv[t * EMB_C + f], (16,))
                return acc + uvv * ivv * wf

            acc = lax.fori_loop(0, EMB_C, fbody, acc)
        z = acc + jnp.broadcast_to(wv[4 * EMB_C], (16,))
        ob[pl.ds(g * 16, 16)] = 1.0 / (1.0 + jnp.exp(-z))
    pltpu.sync_copy(ob, out_hbm.at[pl.ds(base, PPT)])


_score = pl.kernel(
    _score_body,
    out_type=jax.ShapeDtypeStruct((B_C,), jnp.float32),
    mesh=plsc.VectorSubcoreMesh(core_axis_name="c", subcore_axis_name="s"),
    scratch_types=(
        [pltpu.VMEM((PPT,), jnp.int32)] * 2
        + [pltpu.VMEM((PPT, EMB_C), jnp.float32)] * 8
        + [pltpu.VMEM((264,), jnp.float32),
           pltpu.VMEM((PPT,), jnp.float32),
           pltpu.SemaphoreType.DMA]
    ),
)


def kernel(user_emb, item_emb,
           W_gc_0, b_gc_0, W_bi_0, b_bi_0,
           W_gc_1, b_gc_1, W_bi_1, b_bi_1,
           W_gc_2, b_gc_2, W_bi_2, b_bi_2,
           gmf_W, gmf_b, adj_val, adj_row, adj_col,
           user_indices, item_indices):
    Wg = [W_gc_0, W_gc_1, W_gc_2]
    bg = [b_gc_0, b_gc_1, b_gc_2]
    Wb = [W_bi_0, W_bi_1, W_bi_2]
    bb = [b_bi_0, b_bi_1, b_bi_2]

    ego = jnp.concatenate([user_emb, item_emb], axis=0)
    ego0 = ego
    pad = E_PAD - NNZ_C
    cols2d = jnp.pad(adj_col, (0, pad)).reshape(-1, CHUNK)
    rows2d = jnp.pad(adj_row, (0, pad)).reshape(-1, CHUNK)
    vals2d = jnp.pad(adj_val, (0, pad)).reshape(-1, CHUNK)

    norms = []
    for k in range(3):
        egoH = jnp.concatenate([ego[:, :HALF], ego[:, HALF:]], axis=0)
        side2 = _spmm(egoH, cols2d, rows2d, vals2d)
        side = jnp.concatenate([side2[:N_NODES], side2[N_NODES:]], axis=1)
        bg8 = jnp.broadcast_to(bg[k], (8, EMB_C))
        bb8 = jnp.broadcast_to(bb[k], (8, EMB_C))
        ego, norm = _dense(side, ego, Wg[k], bg8, Wb[k], bb8)
        norms.append(norm)

    wvec = jnp.concatenate([gmf_W.reshape(-1), gmf_b.reshape(-1),
                            jnp.zeros((7,), jnp.float32)])
    iidx = item_indices.astype(jnp.int32) + N_USER_C
    uidx = user_indices.astype(jnp.int32)
    out = _score(ego0, norms[0], norms[1], norms[2], uidx, iidx, wvec)
    return out.reshape(B_C, 1)


# speedup vs baseline: 6.0612x; 6.0612x over previous
import jax
import jax.numpy as jnp
from jax import lax
from jax.experimental import pallas as pl
from jax.experimental.pallas import tpu as pltpu
from jax.experimental.pallas import tpu_sc as plsc

N_USER_C = 25000
N_NODES = 50000
EMB_C = 64
HALF = 32
NNZ_C = 800000
B_C = 4096

NTILES = 16
NCORES = 2
CHUNK = 128
CPT = 400
EBLK = 16
NBLK = CPT // EBLK
E_PAD = NTILES * CPT * CHUNK
N_PAD = 50048
ROWS_PER_TILE = N_PAD // NTILES
ZROWS = 136


def _spmm_body(egoL, egoR, cols2d, rows2d, vals2d, out_hbm,
               cols_blk, rows_blk, vals_blk,
               gbuf0, gbuf1, gbuf2, gbuf3, accum, zrow,
               gsem0, gsem1, gsem2, gsem3,
               ssem0, ssem1, ssem2, ssem3):
    c = lax.axis_index("c")
    s = lax.axis_index("s")

    zv = jnp.zeros((16,), jnp.float32)

    def zfill(r, _):
        zrow[r, pl.ds(0, 16)] = zv
        zrow[r, pl.ds(16, 16)] = zv
        return 0

    lax.fori_loop(0, ZROWS, zfill, 0)

    def zloop(k, _):
        pltpu.sync_copy(zrow,
                        accum.at[pl.ds(s * ROWS_PER_TILE + k * ZROWS, ZROWS)])
        return 0

    lax.fori_loop(0, ROWS_PER_TILE // ZROWS, zloop, 0)
    plsc.subcore_barrier()

    gbufs = (gbuf0, gbuf1, gbuf2, gbuf3)
    gsems = (gsem0, gsem1, gsem2, gsem3)
    ssems = (ssem0, ssem1, ssem2, ssem3)

    def start_gather(row, buf):
        @pl.when(c == 0)
        def _():
            pltpu.async_copy(egoL.at[cols_blk.at[row]], gbufs[buf],
                             gsems[buf])

        @pl.when(c == 1)
        def _():
            pltpu.async_copy(egoR.at[cols_blk.at[row]], gbufs[buf],
                             gsems[buf])

    def process(row, buf):
        gb = gbufs[buf]
        pltpu.make_async_copy(
            egoL.at[cols_blk.at[row]], gb, gsems[buf]).wait()

        @plsc.parallel_loop(0, CHUNK // 16, unroll=4)
        def edge16(q):
            vv = vals_blk[row, pl.ds(q * 16, 16)]
            e0 = q * 16
            for t in range(16):
                bv = jnp.broadcast_to(vv[t], (16,))
                gb[e0 + t, pl.ds(0, 16)] = gb[e0 + t, pl.ds(0, 16)] * bv
                gb[e0 + t, pl.ds(16, 16)] = gb[e0 + t, pl.ds(16, 16)] * bv

        pltpu.async_copy(gb, accum.at[rows_blk.at[row]], ssems[buf],
                         add=True)

    def scatter_wait(row, buf):
        pltpu.make_async_copy(gbufs[buf], accum.at[rows_blk.at[row]],
                              ssems[buf]).wait()

    def block(b, _):
        r0 = s * CPT + b * EBLK
        pltpu.sync_copy(cols2d.at[pl.ds(r0, EBLK)], cols_blk)
        pltpu.sync_copy(rows2d.at[pl.ds(r0, EBLK)], rows_blk)
        pltpu.sync_copy(vals2d.at[pl.ds(r0, EBLK)], vals_blk)

        start_gather(0, 0)
        start_gather(1, 1)

        def quad(jj, _):
            for q in range(4):
                j = 4 * jj + q

                @pl.when(j >= 2)
                def _():
                    scatter_wait(j - 2, (q + 2) % 4)

                @pl.when(j < EBLK - 2)
                def _():
                    start_gather(j + 2, (q + 2) % 4)

                process(j, q)
            return 0

        lax.fori_loop(0, EBLK // 4, quad, 0)
        scatter_wait(EBLK - 2, (EBLK - 2) % 4)
        scatter_wait(EBLK - 1, (EBLK - 1) % 4)
        return 0

    lax.fori_loop(0, NBLK, block, 0)

    plsc.subcore_barrier()
    pltpu.sync_copy(
        accum.at[pl.ds(s * ROWS_PER_TILE, ROWS_PER_TILE)],
        out_hbm.at[pl.ds(c * N_PAD + s * ROWS_PER_TILE, ROWS_PER_TILE)])


_spmm = pl.kernel(
    _spmm_body,
    out_type=jax.ShapeDtypeStruct((2 * N_PAD, HALF), jnp.float32),
    mesh=plsc.VectorSubcoreMesh(core_axis_name="c", subcore_axis_name="s"),
    scratch_types=[
        pltpu.VMEM((EBLK, CHUNK), jnp.int32),
        pltpu.VMEM((EBLK, CHUNK), jnp.int32),
        pltpu.VMEM((EBLK, CHUNK), jnp.float32),
        pltpu.VMEM((CHUNK, HALF), jnp.float32),
        pltpu.VMEM((CHUNK, HALF), jnp.float32),
        pltpu.VMEM((CHUNK, HALF), jnp.float32),
        pltpu.VMEM((CHUNK, HALF), jnp.float32),
        pltpu.VMEM_SHARED((N_PAD, HALF), jnp.float32),
        pltpu.VMEM((ZROWS, HALF), jnp.float32),
        pltpu.SemaphoreType.DMA,
        pltpu.SemaphoreType.DMA,
        pltpu.SemaphoreType.DMA,
        pltpu.SemaphoreType.DMA,
        pltpu.SemaphoreType.DMA,
        pltpu.SemaphoreType.DMA,
        pltpu.SemaphoreType.DMA,
        pltpu.SemaphoreType.DMA,
    ],
    compiler_params=pltpu.CompilerParams(use_tc_tiling_on_sc=False),
)


def _dense_body(sl_ref, sr_ref, el_ref, er_ref, wg_ref, bg_ref, wb_ref,
                bb_ref, ol_ref, or_ref, norm_out):
    side = jnp.concatenate([sl_ref[...], sr_ref[...]], axis=1)
    ego = jnp.concatenate([el_ref[...], er_ref[...]], axis=1)
    se = jnp.dot(side, wg_ref[...],
                 preferred_element_type=jnp.float32) + bg_ref[0:1, :]
    bi = jnp.dot(ego * side, wb_ref[...],
                 preferred_element_type=jnp.float32) + bb_ref[0:1, :]
    x = se + bi
    x = jnp.where(x >= 0, x, 0.2 * x)
    nrm = jnp.sqrt(jnp.sum(x * x, axis=1, keepdims=True))
    norm_out[...] = x / jnp.maximum(nrm, 1e-12)
    ol_ref[...] = x[:, :HALF]
    or_ref[...] = x[:, HALF:]


def _dense(side2, egoL, egoR, wg, bg8, wb, bb8):
    R = 3128
    nb = N_PAD // R
    return pl.pallas_call(
        _dense_body,
        grid=(nb,),
        in_specs=[
            pl.BlockSpec((R, HALF), lambda i: (i, 0)),
            pl.BlockSpec((R, HALF), lambda i: (i + 16, 0)),
            pl.BlockSpec((R, HALF), lambda i: (i, 0)),
            pl.BlockSpec((R, HALF), lambda i: (i, 0)),
            pl.BlockSpec((EMB_C, EMB_C), lambda i: (0, 0)),
            pl.BlockSpec((8, EMB_C), lambda i: (0, 0)),
            pl.BlockSpec((EMB_C, EMB_C), lambda i: (0, 0)),
            pl.BlockSpec((8, EMB_C), lambda i: (0, 0)),
        ],
        out_specs=[
            pl.BlockSpec((R, HALF), lambda i: (i, 0)),
            pl.BlockSpec((R, HALF), lambda i: (i, 0)),
            pl.BlockSpec((R, EMB_C), lambda i: (i, 0)),
        ],
        out_shape=[
            jax.ShapeDtypeStruct((N_PAD, HALF), jnp.float32),
            jax.ShapeDtypeStruct((N_PAD, HALF), jnp.float32),
            jax.ShapeDtypeStruct((N_PAD, EMB_C), jnp.float32),
        ],
    )(side2, side2, egoL, egoR, wg, bg8, wb, bb8)


PPT = B_C // (NTILES * NCORES)


def _gather_body(t0, t1, t2, t3, uidx, iidx, ug_hbm, ig_hbm,
                 uix, iix, gb, sem):
    c = lax.axis_index("c")
    s = lax.axis_index("s")
    wid = s * NCORES + c
    base = wid * PPT
    pltpu.sync_copy(uidx.at[pl.ds(base, PPT)], uix)
    pltpu.sync_copy(iidx.at[pl.ds(base, PPT)], iix)
    for t, tab in enumerate([t0, t1, t2, t3]):
        pltpu.async_copy(tab.at[uix], gb, sem).wait()
        pltpu.sync_copy(gb, ug_hbm.at[t, pl.ds(base, PPT)])
        pltpu.async_copy(tab.at[iix], gb, sem).wait()
        pltpu.sync_copy(gb, ig_hbm.at[t, pl.ds(base, PPT)])


_gather = pl.kernel(
    _gather_body,
    out_type=[jax.ShapeDtypeStruct((4, B_C, EMB_C), jnp.float32)] * 2,
    mesh=plsc.VectorSubcoreMesh(core_axis_name="c", subcore_axis_name="s"),
    scratch_types=[
        pltpu.VMEM((PPT,), jnp.int32),
        pltpu.VMEM((PPT,), jnp.int32),
        pltpu.VMEM((PPT, EMB_C), jnp.float32),
        pltpu.SemaphoreType.DMA,
    ],
    compiler_params=pltpu.CompilerParams(use_tc_tiling_on_sc=False),
)


def _score_tc_body(ug_ref, ig_ref, wb_ref, out_ref):
    acc = jnp.zeros((out_ref.shape[0], 1), jnp.float32)
    for t in range(4):
        u = ug_ref[t, :, :]
        i = ig_ref[t, :, :]
        w_t = wb_ref[0:1, t * EMB_C:(t + 1) * EMB_C]
        acc = acc + jnp.sum(u * i * w_t, axis=1, keepdims=True)
    z = acc + wb_ref[1:2, 0:1]
    out_ref[...] = jax.nn.sigmoid(z)


def _score_tc(ug, ig, wb8):
    R = 512
    nb = B_C // R
    return pl.pallas_call(
        _score_tc_body,
        grid=(nb,),
        in_specs=[
            pl.BlockSpec((4, R, EMB_C), lambda i: (0, i, 0)),
            pl.BlockSpec((4, R, EMB_C), lambda i: (0, i, 0)),
            pl.BlockSpec((8, 4 * EMB_C), lambda i: (0, 0)),
        ],
        out_specs=pl.BlockSpec((R, 1), lambda i: (i, 0)),
        out_shape=jax.ShapeDtypeStruct((B_C, 1), jnp.float32),
    )(ug, ig, wb8)


def kernel(user_emb, item_emb,
           W_gc_0, b_gc_0, W_bi_0, b_bi_0,
           W_gc_1, b_gc_1, W_bi_1, b_bi_1,
           W_gc_2, b_gc_2, W_bi_2, b_bi_2,
           gmf_W, gmf_b, adj_val, adj_row, adj_col,
           user_indices, item_indices):
    Wg = [W_gc_0, W_gc_1, W_gc_2]
    bg = [b_gc_0, b_gc_1, b_gc_2]
    Wb = [W_bi_0, W_bi_1, W_bi_2]
    bb = [b_bi_0, b_bi_1, b_bi_2]

    ego0 = jnp.concatenate([user_emb, item_emb], axis=0)
    ego0p = jnp.pad(ego0, ((0, N_PAD - N_NODES), (0, 0)))
    egoL = ego0p[:, :HALF]
    egoR = ego0p[:, HALF:]
    pad = E_PAD - NNZ_C
    cols2d = jnp.pad(adj_col, (0, pad)).reshape(-1, CHUNK)
    rows2d = jnp.pad(adj_row, (0, pad)).reshape(-1, CHUNK)
    vals2d = jnp.pad(adj_val, (0, pad)).reshape(-1, CHUNK)

    norms = []
    for k in range(3):
        side2 = _spmm(egoL, egoR, cols2d, rows2d, vals2d)
        bg8 = jnp.broadcast_to(bg[k], (8, EMB_C))
        bb8 = jnp.broadcast_to(bb[k], (8, EMB_C))
        egoL, egoR, norm = _dense(side2, egoL, egoR, Wg[k], bg8, Wb[k], bb8)
        norms.append(norm)

    wb8 = jnp.zeros((8, 4 * EMB_C), jnp.float32)
    wb8 = wb8.at[0].set(gmf_W.reshape(-1)).at[1, 0].set(gmf_b[0])
    iidx = item_indices.astype(jnp.int32) + N_USER_C
    uidx = user_indices.astype(jnp.int32)
    ug, ig = _gather(ego0p, norms[0], norms[1], norms[2], uidx, iidx)
    return _score_tc(ug, ig, wb8)


# speedup vs baseline: 6.6518x; 1.0974x over previous
import jax
import jax.numpy as jnp
from jax import lax
from jax.experimental import pallas as pl
from jax.experimental.pallas import tpu as pltpu
from jax.experimental.pallas import tpu_sc as plsc

N_USER_C = 25000
N_NODES = 50000
EMB_C = 64
HALF = 32
NNZ_C = 800000
B_C = 4096

NTILES = 16
NCORES = 2
CHUNK = 128
CPT = 400
EBLK = 8
NBLK = CPT // EBLK
NPAIR = NBLK // 2
E_PAD = NTILES * CPT * CHUNK
N_PAD = 50048
ROWS_PER_TILE = N_PAD // NTILES
ZROWS = 136


def _spmm_body(egoL, egoR, cols2d, rows2d, vals2d, out_hbm,
               colsA, rowsA, valsA, colsB, rowsB, valsB,
               gbuf0, gbuf1, gbuf2, gbuf3, accum, zrow,
               gsem0, gsem1, gsem2, gsem3,
               ssem0, ssem1, ssem2, ssem3, esem):
    c = lax.axis_index("c")
    s = lax.axis_index("s")

    zv = jnp.zeros((16,), jnp.float32)

    def zfill(r, _):
        zrow[r, pl.ds(0, 16)] = zv
        zrow[r, pl.ds(16, 16)] = zv
        return 0

    lax.fori_loop(0, ZROWS, zfill, 0)

    def zloop(k, _):
        pltpu.sync_copy(zrow,
                        accum.at[pl.ds(s * ROWS_PER_TILE + k * ZROWS, ZROWS)])
        return 0

    lax.fori_loop(0, ROWS_PER_TILE // ZROWS, zloop, 0)
    plsc.subcore_barrier()

    gbufs = (gbuf0, gbuf1, gbuf2, gbuf3)
    gsems = (gsem0, gsem1, gsem2, gsem3)
    ssems = (ssem0, ssem1, ssem2, ssem3)
    ebufA = (colsA, rowsA, valsA)
    ebufB = (colsB, rowsB, valsB)

    def stage_start(b, ebuf):
        r0 = s * CPT + b * EBLK
        pltpu.async_copy(cols2d.at[pl.ds(r0, EBLK)], ebuf[0], esem)
        pltpu.async_copy(rows2d.at[pl.ds(r0, EBLK)], ebuf[1], esem)
        pltpu.async_copy(vals2d.at[pl.ds(r0, EBLK)], ebuf[2], esem)

    def stage_wait(b, ebuf):
        r0 = s * CPT + b * EBLK
        pltpu.make_async_copy(cols2d.at[pl.ds(r0, EBLK)], ebuf[0],
                              esem).wait()
        pltpu.make_async_copy(rows2d.at[pl.ds(r0, EBLK)], ebuf[1],
                              esem).wait()
        pltpu.make_async_copy(vals2d.at[pl.ds(r0, EBLK)], ebuf[2],
                              esem).wait()

    def start_gather(cols_r, row, buf):
        @pl.when(c == 0)
        def _():
            pltpu.async_copy(egoL.at[cols_r.at[row]], gbufs[buf],
                             gsems[buf])

        @pl.when(c == 1)
        def _():
            pltpu.async_copy(egoR.at[cols_r.at[row]], gbufs[buf],
                             gsems[buf])

    def process(cols_r, rows_r, vals_r, row, buf):
        gb = gbufs[buf]
        pltpu.make_async_copy(
            egoL.at[cols_r.at[row]], gb, gsems[buf]).wait()

        @plsc.parallel_loop(0, CHUNK // 16, unroll=2)
        def edge16(q):
            vv = vals_r[row, pl.ds(q * 16, 16)]
            e0 = q * 16
            for t in range(16):
                bv = jnp.broadcast_to(vv[t], (16,))
                gb[e0 + t, pl.ds(0, 16)] = gb[e0 + t, pl.ds(0, 16)] * bv
                gb[e0 + t, pl.ds(16, 16)] = gb[e0 + t, pl.ds(16, 16)] * bv

        pltpu.async_copy(gb, accum.at[rows_r.at[row]], ssems[buf],
                         add=True)

    def scatter_wait(rows_r, row, buf):
        pltpu.make_async_copy(gbufs[buf], accum.at[rows_r.at[row]],
                              ssems[buf]).wait()

    def do_block(b, cur, nxt, first, last):
        colsC, rowsC, valsC = cur
        colsN, rowsN, valsN = nxt
        for j in range(EBLK):
            buf = j % 4
            if j >= 2:
                scatter_wait(rowsC, j - 2, (j + 2) % 4)
            elif first is not True:
                @pl.when(first == False)
                def _():
                    scatter_wait(rowsN, j + EBLK - 2, (j + 2) % 4)

            if j == 2:
                @pl.when(last == False)
                def _():
                    stage_start(b + 1, nxt)

            if j < EBLK - 2:
                start_gather(colsC, j + 2, (j + 2) % 4)
            elif j == EBLK - 2:
                @pl.when(last == False)
                def _():
                    stage_wait(b + 1, nxt)
                    start_gather(colsN, 0, (j + 2) % 4)
            else:
                @pl.when(last == False)
                def _():
                    start_gather(colsN, 1, (j + 2) % 4)

            process(colsC, rowsC, valsC, j, buf)

    stage_start(0, ebufA)
    stage_wait(0, ebufA)
    start_gather(colsA, 0, 0)
    start_gather(colsA, 1, 1)

    def pairloop(bb, _):
        b0 = 2 * bb
        do_block(b0, ebufA, ebufB, first=(bb == 0), last=False)
        do_block(b0 + 1, ebufB, ebufA, first=False,
                 last=(bb == NPAIR - 1))
        return 0

    lax.fori_loop(0, NPAIR, pairloop, 0)
    scatter_wait(rowsB, EBLK - 2, (EBLK - 2) % 4)
    scatter_wait(rowsB, EBLK - 1, (EBLK - 1) % 4)

    plsc.subcore_barrier()
    pltpu.sync_copy(
        accum.at[pl.ds(s * ROWS_PER_TILE, ROWS_PER_TILE)],
        out_hbm.at[pl.ds(c * N_PAD + s * ROWS_PER_TILE, ROWS_PER_TILE)])


_spmm = pl.kernel(
    _spmm_body,
    out_type=jax.ShapeDtypeStruct((2 * N_PAD, HALF), jnp.float32),
    mesh=plsc.VectorSubcoreMesh(core_axis_name="c", subcore_axis_name="s"),
    scratch_types=[
        pltpu.VMEM((EBLK, CHUNK), jnp.int32),
        pltpu.VMEM((EBLK, CHUNK), jnp.int32),
        pltpu.VMEM((EBLK, CHUNK), jnp.float32),
        pltpu.VMEM((EBLK, CHUNK), jnp.int32),
        pltpu.VMEM((EBLK, CHUNK), jnp.int32),
        pltpu.VMEM((EBLK, CHUNK), jnp.float32),
        pltpu.VMEM((CHUNK, HALF), jnp.float32),
        pltpu.VMEM((CHUNK, HALF), jnp.float32),
        pltpu.VMEM((CHUNK, HALF), jnp.float32),
        pltpu.VMEM((CHUNK, HALF), jnp.float32),
        pltpu.VMEM_SHARED((N_PAD, HALF), jnp.float32),
        pltpu.VMEM((ZROWS, HALF), jnp.float32),
        pltpu.SemaphoreType.DMA,
        pltpu.SemaphoreType.DMA,
        pltpu.SemaphoreType.DMA,
        pltpu.SemaphoreType.DMA,
        pltpu.SemaphoreType.DMA,
        pltpu.SemaphoreType.DMA,
        pltpu.SemaphoreType.DMA,
        pltpu.SemaphoreType.DMA,
        pltpu.SemaphoreType.DMA,
    ],
    compiler_params=pltpu.CompilerParams(use_tc_tiling_on_sc=False),
)


def _dense_body(sl_ref, sr_ref, el_ref, er_ref, wg_ref, bg_ref, wb_ref,
                bb_ref, ol_ref, or_ref, norm_out):
    side = jnp.concatenate([sl_ref[...], sr_ref[...]], axis=1)
    ego = jnp.concatenate([el_ref[...], er_ref[...]], axis=1)
    se = jnp.dot(side, wg_ref[...],
                 preferred_element_type=jnp.float32) + bg_ref[0:1, :]
    bi = jnp.dot(ego * side, wb_ref[...],
                 preferred_element_type=jnp.float32) + bb_ref[0:1, :]
    x = se + bi
    x = jnp.where(x >= 0, x, 0.2 * x)
    nrm = jnp.sqrt(jnp.sum(x * x, axis=1, keepdims=True))
    norm_out[...] = x / jnp.maximum(nrm, 1e-12)
    ol_ref[...] = x[:, :HALF]
    or_ref[...] = x[:, HALF:]


def _dense(side2, egoL, egoR, wg, bg8, wb, bb8):
    R = 3128
    nb = N_PAD // R
    return pl.pallas_call(
        _dense_body,
        grid=(nb,),
        in_specs=[
            pl.BlockSpec((R, HALF), lambda i: (i, 0)),
            pl.BlockSpec((R, HALF), lambda i: (i + 16, 0)),
            pl.BlockSpec((R, HALF), lambda i: (i, 0)),
            pl.BlockSpec((R, HALF), lambda i: (i, 0)),
            pl.BlockSpec((EMB_C, EMB_C), lambda i: (0, 0)),
            pl.BlockSpec((8, EMB_C), lambda i: (0, 0)),
            pl.BlockSpec((EMB_C, EMB_C), lambda i: (0, 0)),
            pl.BlockSpec((8, EMB_C), lambda i: (0, 0)),
        ],
        out_specs=[
            pl.BlockSpec((R, HALF), lambda i: (i, 0)),
            pl.BlockSpec((R, HALF), lambda i: (i, 0)),
            pl.BlockSpec((R, EMB_C), lambda i: (i, 0)),
        ],
        out_shape=[
            jax.ShapeDtypeStruct((N_PAD, HALF), jnp.float32),
            jax.ShapeDtypeStruct((N_PAD, HALF), jnp.float32),
            jax.ShapeDtypeStruct((N_PAD, EMB_C), jnp.float32),
        ],
    )(side2, side2, egoL, egoR, wg, bg8, wb, bb8)


PPT = B_C // (NTILES * NCORES)


def _gather_body(t0, t1, t2, t3, uidx, iidx, ug_hbm, ig_hbm,
                 uix, iix, gb, sem):
    c = lax.axis_index("c")
    s = lax.axis_index("s")
    wid = s * NCORES + c
    base = wid * PPT
    pltpu.sync_copy(uidx.at[pl.ds(base, PPT)], uix)
    pltpu.sync_copy(iidx.at[pl.ds(base, PPT)], iix)
    for t, tab in enumerate([t0, t1, t2, t3]):
        pltpu.async_copy(tab.at[uix], gb, sem).wait()
        pltpu.sync_copy(gb, ug_hbm.at[t, pl.ds(base, PPT)])
        pltpu.async_copy(tab.at[iix], gb, sem).wait()
        pltpu.sync_copy(gb, ig_hbm.at[t, pl.ds(base, PPT)])


_gather = pl.kernel(
    _gather_body,
    out_type=[jax.ShapeDtypeStruct((4, B_C, EMB_C), jnp.float32)] * 2,
    mesh=plsc.VectorSubcoreMesh(core_axis_name="c", subcore_axis_name="s"),
    scratch_types=[
        pltpu.VMEM((PPT,), jnp.int32),
        pltpu.VMEM((PPT,), jnp.int32),
        pltpu.VMEM((PPT, EMB_C), jnp.float32),
        pltpu.SemaphoreType.DMA,
    ],
    compiler_params=pltpu.CompilerParams(use_tc_tiling_on_sc=False),
)


def _score_tc_body(ug_ref, ig_ref, wb_ref, out_ref):
    acc = jnp.zeros((out_ref.shape[0], 1), jnp.float32)
    for t in range(4):
        u = ug_ref[t, :, :]
        i = ig_ref[t, :, :]
        w_t = wb_ref[0:1, t * EMB_C:(t + 1) * EMB_C]
        acc = acc + jnp.sum(u * i * w_t, axis=1, keepdims=True)
    z = acc + wb_ref[1:2, 0:1]
    out_ref[...] = jax.nn.sigmoid(z)


def _score_tc(ug, ig, wb8):
    R = 512
    nb = B_C // R
    return pl.pallas_call(
        _score_tc_body,
        grid=(nb,),
        in_specs=[
            pl.BlockSpec((4, R, EMB_C), lambda i: (0, i, 0)),
            pl.BlockSpec((4, R, EMB_C), lambda i: (0, i, 0)),
            pl.BlockSpec((8, 4 * EMB_C), lambda i: (0, 0)),
        ],
        out_specs=pl.BlockSpec((R, 1), lambda i: (i, 0)),
        out_shape=jax.ShapeDtypeStruct((B_C, 1), jnp.float32),
    )(ug, ig, wb8)


def kernel(user_emb, item_emb,
           W_gc_0, b_gc_0, W_bi_0, b_bi_0,
           W_gc_1, b_gc_1, W_bi_1, b_bi_1,
           W_gc_2, b_gc_2, W_bi_2, b_bi_2,
           gmf_W, gmf_b, adj_val, adj_row, adj_col,
           user_indices, item_indices):
    Wg = [W_gc_0, W_gc_1, W_gc_2]
    bg = [b_gc_0, b_gc_1, b_gc_2]
    Wb = [W_bi_0, W_bi_1, W_bi_2]
    bb = [b_bi_0, b_bi_1, b_bi_2]

    ego0 = jnp.concatenate([user_emb, item_emb], axis=0)
    ego0p = jnp.pad(ego0, ((0, N_PAD - N_NODES), (0, 0)))
    egoL = ego0p[:, :HALF]
    egoR = ego0p[:, HALF:]
    pad = E_PAD - NNZ_C
    cols2d = jnp.pad(adj_col, (0, pad)).reshape(-1, CHUNK)
    rows2d = jnp.pad(adj_row, (0, pad)).reshape(-1, CHUNK)
    vals2d = jnp.pad(adj_val, (0, pad)).reshape(-1, CHUNK)

    norms = []
    for k in range(3):
        side2 = _spmm(egoL, egoR, cols2d, rows2d, vals2d)
        bg8 = jnp.broadcast_to(bg[k], (8, EMB_C))
        bb8 = jnp.broadcast_to(bb[k], (8, EMB_C))
        egoL, egoR, norm = _dense(side2, egoL, egoR, Wg[k], bg8, Wb[k], bb8)
        norms.append(norm)

    wb8 = jnp.zeros((8, 4 * EMB_C), jnp.float32)
    wb8 = wb8.at[0].set(gmf_W.reshape(-1)).at[1, 0].set(gmf_b[0])
    iidx = item_indices.astype(jnp.int32) + N_USER_C
    uidx = user_indices.astype(jnp.int32)
    ug, ig = _gather(ego0p, norms[0], norms[1], norms[2], uidx, iidx)
    return _score_tc(ug, ig, wb8)


# speedup vs baseline: 6.6884x; 1.0055x over previous
import jax
import jax.numpy as jnp
from jax import lax
from jax.experimental import pallas as pl
from jax.experimental.pallas import tpu as pltpu
from jax.experimental.pallas import tpu_sc as plsc

N_USER_C = 25000
N_NODES = 50000
EMB_C = 64
HALF = 32
NNZ_C = 800000
B_C = 4096

NTILES = 16
NCORES = 2
CHUNK = 128
CPT = 400
EBLK = 8
NBLK = CPT // EBLK
NPAIR = NBLK // 2
E_PAD = NTILES * CPT * CHUNK
N_PAD = 50048
ROWS_PER_TILE = N_PAD // NTILES
ZROWS = 136


def _spmm_body(egoL, egoR, cols2d, rows2d, vals2d, out_hbm,
               colsA, rowsA, valsA, colsB, rowsB, valsB,
               gbuf0, gbuf1, gbuf2, gbuf3, accum, zrow,
               gsem0, gsem1, gsem2, gsem3,
               ssem0, ssem1, ssem2, ssem3, esem):
    c = lax.axis_index("c")
    s = lax.axis_index("s")

    zv = jnp.zeros((16,), jnp.float32)

    def zfill(r, _):
        zrow[r, pl.ds(0, 16)] = zv
        zrow[r, pl.ds(16, 16)] = zv
        return 0

    lax.fori_loop(0, ZROWS, zfill, 0)

    def zloop(k, _):
        pltpu.sync_copy(zrow,
                        accum.at[pl.ds(s * ROWS_PER_TILE + k * ZROWS, ZROWS)])
        return 0

    lax.fori_loop(0, ROWS_PER_TILE // ZROWS, zloop, 0)
    plsc.subcore_barrier()

    gbufs = (gbuf0, gbuf1, gbuf2, gbuf3)
    gsems = (gsem0, gsem1, gsem2, gsem3)
    ssems = (ssem0, ssem1, ssem2, ssem3)
    ebufA = (colsA, rowsA, valsA)
    ebufB = (colsB, rowsB, valsB)

    def stage_start(b, ebuf):
        r0 = s * CPT + b * EBLK
        pltpu.async_copy(cols2d.at[pl.ds(r0, EBLK)], ebuf[0], esem)
        pltpu.async_copy(rows2d.at[pl.ds(r0, EBLK)], ebuf[1], esem)
        pltpu.async_copy(vals2d.at[pl.ds(r0, EBLK)], ebuf[2], esem)

    def stage_wait(b, ebuf):
        r0 = s * CPT + b * EBLK
        pltpu.make_async_copy(cols2d.at[pl.ds(r0, EBLK)], ebuf[0],
                              esem).wait()
        pltpu.make_async_copy(rows2d.at[pl.ds(r0, EBLK)], ebuf[1],
                              esem).wait()
        pltpu.make_async_copy(vals2d.at[pl.ds(r0, EBLK)], ebuf[2],
                              esem).wait()

    def start_gather(cols_r, row, buf):
        @pl.when(c == 0)
        def _():
            pltpu.async_copy(egoL.at[cols_r.at[row]], gbufs[buf],
                             gsems[buf])

        @pl.when(c == 1)
        def _():
            pltpu.async_copy(egoR.at[cols_r.at[row]], gbufs[buf],
                             gsems[buf])

    def process(cols_r, rows_r, vals_r, row, buf):
        gb = gbufs[buf]
        pltpu.make_async_copy(
            egoL.at[cols_r.at[row]], gb, gsems[buf]).wait()

        @plsc.parallel_loop(0, CHUNK // 16, unroll=2)
        def edge16(q):
            vv = vals_r[row, pl.ds(q * 16, 16)]
            e0 = q * 16
            for t in range(16):
                bv = jnp.broadcast_to(vv[t], (16,))
                gb[e0 + t, pl.ds(0, 16)] = gb[e0 + t, pl.ds(0, 16)] * bv
                gb[e0 + t, pl.ds(16, 16)] = gb[e0 + t, pl.ds(16, 16)] * bv

        pltpu.async_copy(gb, accum.at[rows_r.at[row]], ssems[buf],
                         add=True)

    def scatter_wait(rows_r, row, buf):
        pltpu.make_async_copy(gbufs[buf], accum.at[rows_r.at[row]],
                              ssems[buf]).wait()

    def do_block(b, cur, nxt, first, last):
        colsC, rowsC, valsC = cur
        colsN, rowsN, valsN = nxt
        for j in range(EBLK):
            buf = j % 4
            if j >= 2:
                scatter_wait(rowsC, j - 2, (j + 2) % 4)
            elif first is not True:
                @pl.when(first == False)
                def _():
                    scatter_wait(rowsN, j + EBLK - 2, (j + 2) % 4)

            if j == 2:
                @pl.when(last == False)
                def _():
                    stage_start(b + 1, nxt)

            if j < EBLK - 2:
                start_gather(colsC, j + 2, (j + 2) % 4)
            elif j == EBLK - 2:
                @pl.when(last == False)
                def _():
                    stage_wait(b + 1, nxt)
                    start_gather(colsN, 0, (j + 2) % 4)
            else:
                @pl.when(last == False)
                def _():
                    start_gather(colsN, 1, (j + 2) % 4)

            process(colsC, rowsC, valsC, j, buf)

    stage_start(0, ebufA)
    stage_wait(0, ebufA)
    start_gather(colsA, 0, 0)
    start_gather(colsA, 1, 1)

    def pairloop(bb, _):
        b0 = 2 * bb
        do_block(b0, ebufA, ebufB, first=(bb == 0), last=False)
        do_block(b0 + 1, ebufB, ebufA, first=False,
                 last=(bb == NPAIR - 1))
        return 0

    lax.fori_loop(0, NPAIR, pairloop, 0)
    scatter_wait(rowsB, EBLK - 2, (EBLK - 2) % 4)
    scatter_wait(rowsB, EBLK - 1, (EBLK - 1) % 4)

    plsc.subcore_barrier()
    pltpu.sync_copy(
        accum.at[pl.ds(s * ROWS_PER_TILE, ROWS_PER_TILE)],
        out_hbm.at[pl.ds(c * N_PAD + s * ROWS_PER_TILE, ROWS_PER_TILE)])


_spmm = pl.kernel(
    _spmm_body,
    out_type=jax.ShapeDtypeStruct((2 * N_PAD, HALF), jnp.float32),
    mesh=plsc.VectorSubcoreMesh(core_axis_name="c", subcore_axis_name="s"),
    scratch_types=[
        pltpu.VMEM((EBLK, CHUNK), jnp.int32),
        pltpu.VMEM((EBLK, CHUNK), jnp.int32),
        pltpu.VMEM((EBLK, CHUNK), jnp.float32),
        pltpu.VMEM((EBLK, CHUNK), jnp.int32),
        pltpu.VMEM((EBLK, CHUNK), jnp.int32),
        pltpu.VMEM((EBLK, CHUNK), jnp.float32),
        pltpu.VMEM((CHUNK, HALF), jnp.float32),
        pltpu.VMEM((CHUNK, HALF), jnp.float32),
        pltpu.VMEM((CHUNK, HALF), jnp.float32),
        pltpu.VMEM((CHUNK, HALF), jnp.float32),
        pltpu.VMEM_SHARED((N_PAD, HALF), jnp.float32),
        pltpu.VMEM((ZROWS, HALF), jnp.float32),
        pltpu.SemaphoreType.DMA,
        pltpu.SemaphoreType.DMA,
        pltpu.SemaphoreType.DMA,
        pltpu.SemaphoreType.DMA,
        pltpu.SemaphoreType.DMA,
        pltpu.SemaphoreType.DMA,
        pltpu.SemaphoreType.DMA,
        pltpu.SemaphoreType.DMA,
        pltpu.SemaphoreType.DMA,
    ],
    compiler_params=pltpu.CompilerParams(use_tc_tiling_on_sc=False),
)


def _dense_body(sl_ref, sr_ref, el_ref, er_ref, wg_ref, bg_ref, wb_ref,
                bb_ref, ol_ref, or_ref, norm_out):
    side = jnp.concatenate([sl_ref[...], sr_ref[...]], axis=1)
    ego = jnp.concatenate([el_ref[...], er_ref[...]], axis=1)
    se = jnp.dot(side, wg_ref[...],
                 preferred_element_type=jnp.float32) + bg_ref[0:1, :]
    bi = jnp.dot(ego * side, wb_ref[...],
                 preferred_element_type=jnp.float32) + bb_ref[0:1, :]
    x = se + bi
    x = jnp.where(x >= 0, x, 0.2 * x)
    nrm = jnp.sqrt(jnp.sum(x * x, axis=1, keepdims=True))
    norm_out[...] = x / jnp.maximum(nrm, 1e-12)
    ol_ref[...] = x[:, :HALF]
    or_ref[...] = x[:, HALF:]


def _dense(side2, egoL, egoR, wg, bg8, wb, bb8):
    R = 6256
    nb = N_PAD // R
    return pl.pallas_call(
        _dense_body,
        grid=(nb,),
        in_specs=[
            pl.BlockSpec((R, HALF), lambda i: (i, 0)),
            pl.BlockSpec((R, HALF), lambda i: (i + 8, 0)),
            pl.BlockSpec((R, HALF), lambda i: (i, 0)),
            pl.BlockSpec((R, HALF), lambda i: (i, 0)),
            pl.BlockSpec((EMB_C, EMB_C), lambda i: (0, 0)),
            pl.BlockSpec((8, EMB_C), lambda i: (0, 0)),
            pl.BlockSpec((EMB_C, EMB_C), lambda i: (0, 0)),
            pl.BlockSpec((8, EMB_C), lambda i: (0, 0)),
        ],
        out_specs=[
            pl.BlockSpec((R, HALF), lambda i: (i, 0)),
            pl.BlockSpec((R, HALF), lambda i: (i, 0)),
            pl.BlockSpec((R, EMB_C), lambda i: (i, 0)),
        ],
        out_shape=[
            jax.ShapeDtypeStruct((N_PAD, HALF), jnp.float32),
            jax.ShapeDtypeStruct((N_PAD, HALF), jnp.float32),
            jax.ShapeDtypeStruct((N_PAD, EMB_C), jnp.float32),
        ],
    )(side2, side2, egoL, egoR, wg, bg8, wb, bb8)


PPT = B_C // (NTILES * NCORES)


def _gather_body(t0, t1, t2, t3, uidx, iidx, ug_hbm, ig_hbm,
                 uix, iix, gb, sem):
    c = lax.axis_index("c")
    s = lax.axis_index("s")
    wid = s * NCORES + c
    base = wid * PPT
    pltpu.sync_copy(uidx.at[pl.ds(base, PPT)], uix)
    pltpu.sync_copy(iidx.at[pl.ds(base, PPT)], iix)
    for t, tab in enumerate([t0, t1, t2, t3]):
        pltpu.async_copy(tab.at[uix], gb, sem).wait()
        pltpu.sync_copy(gb, ug_hbm.at[t, pl.ds(base, PPT)])
        pltpu.async_copy(tab.at[iix], gb, sem).wait()
        pltpu.sync_copy(gb, ig_hbm.at[t, pl.ds(base, PPT)])


_gather = pl.kernel(
    _gather_body,
    out_type=[jax.ShapeDtypeStruct((4, B_C, EMB_C), jnp.float32)] * 2,
    mesh=plsc.VectorSubcoreMesh(core_axis_name="c", subcore_axis_name="s"),
    scratch_types=[
        pltpu.VMEM((PPT,), jnp.int32),
        pltpu.VMEM((PPT,), jnp.int32),
        pltpu.VMEM((PPT, EMB_C), jnp.float32),
        pltpu.SemaphoreType.DMA,
    ],
    compiler_params=pltpu.CompilerParams(use_tc_tiling_on_sc=False),
)


def _score_tc_body(ug_ref, ig_ref, wb_ref, out_ref):
    acc = jnp.zeros((out_ref.shape[0], 1), jnp.float32)
    for t in range(4):
        u = ug_ref[t, :, :]
        i = ig_ref[t, :, :]
        w_t = wb_ref[0:1, t * EMB_C:(t + 1) * EMB_C]
        acc = acc + jnp.sum(u * i * w_t, axis=1, keepdims=True)
    z = acc + wb_ref[1:2, 0:1]
    out_ref[...] = jax.nn.sigmoid(z)


def _score_tc(ug, ig, wb8):
    R = 512
    nb = B_C // R
    return pl.pallas_call(
        _score_tc_body,
        grid=(nb,),
        in_specs=[
            pl.BlockSpec((4, R, EMB_C), lambda i: (0, i, 0)),
            pl.BlockSpec((4, R, EMB_C), lambda i: (0, i, 0)),
            pl.BlockSpec((8, 4 * EMB_C), lambda i: (0, 0)),
        ],
        out_specs=pl.BlockSpec((R, 1), lambda i: (i, 0)),
        out_shape=jax.ShapeDtypeStruct((B_C, 1), jnp.float32),
    )(ug, ig, wb8)


def kernel(user_emb, item_emb,
           W_gc_0, b_gc_0, W_bi_0, b_bi_0,
           W_gc_1, b_gc_1, W_bi_1, b_bi_1,
           W_gc_2, b_gc_2, W_bi_2, b_bi_2,
           gmf_W, gmf_b, adj_val, adj_row, adj_col,
           user_indices, item_indices):
    Wg = [W_gc_0, W_gc_1, W_gc_2]
    bg = [b_gc_0, b_gc_1, b_gc_2]
    Wb = [W_bi_0, W_bi_1, W_bi_2]
    bb = [b_bi_0, b_bi_1, b_bi_2]

    ego0 = jnp.concatenate([user_emb, item_emb], axis=0)
    ego0p = jnp.pad(ego0, ((0, N_PAD - N_NODES), (0, 0)))
    egoL = ego0p[:, :HALF]
    egoR = ego0p[:, HALF:]
    pad = E_PAD - NNZ_C
    cols2d = jnp.pad(adj_col, (0, pad)).reshape(-1, CHUNK)
    rows2d = jnp.pad(adj_row, (0, pad)).reshape(-1, CHUNK)
    vals2d = jnp.pad(adj_val, (0, pad)).reshape(-1, CHUNK)

    norms = []
    for k in range(3):
        side2 = _spmm(egoL, egoR, cols2d, rows2d, vals2d)
        bg8 = jnp.broadcast_to(bg[k], (8, EMB_C))
        bb8 = jnp.broadcast_to(bb[k], (8, EMB_C))
        egoL, egoR, norm = _dense(side2, egoL, egoR, Wg[k], bg8, Wb[k], bb8)
        norms.append(norm)

    wb8 = jnp.zeros((8, 4 * EMB_C), jnp.float32)
    wb8 = wb8.at[0].set(gmf_W.reshape(-1)).at[1, 0].set(gmf_b[0])
    iidx = item_indices.astype(jnp.int32) + N_USER_C
    uidx = user_indices.astype(jnp.int32)
    ug, ig = _gather(ego0p, norms[0], norms[1], norms[2], uidx, iidx)
    return _score_tc(ug, ig, wb8)
